# baseline (speedup 1.0000x reference)
import jax
import jax.numpy as jnp
from jax.experimental import pallas as pl
from jax.experimental.pallas import tpu as pltpu

_ROWS = 8192
_COLS = 128
_CHUNK_ROWS = (512, 1024, 2048, 4608)
_OFFSETS = (0, 512, 1536, 3584)
_N_CHUNKS = len(_CHUNK_ROWS)


def _copy_pipeline(t_hbm, o_hbm, buf, *sems):
    in_sems = sems[:_N_CHUNKS]
    out_sems = sems[_N_CHUNKS:]
    ins = [
        pltpu.make_async_copy(
            t_hbm.at[pl.ds(_OFFSETS[c], _CHUNK_ROWS[c]), :],
            buf.at[pl.ds(_OFFSETS[c], _CHUNK_ROWS[c]), :],
            in_sems[c],
        )
        for c in range(_N_CHUNKS)
    ]
    outs = [
        pltpu.make_async_copy(
            buf.at[pl.ds(_OFFSETS[c], _CHUNK_ROWS[c]), :],
            o_hbm.at[pl.ds(_OFFSETS[c], _CHUNK_ROWS[c]), :],
            out_sems[c],
        )
        for c in range(_N_CHUNKS)
    ]
    for c in range(_N_CHUNKS):
        ins[c].start()
    for c in range(_N_CHUNKS):
        ins[c].wait()
        outs[c].start()
    for c in range(_N_CHUNKS):
        outs[c].wait()


def kernel(x, pos_table):
    del x
    return pl.pallas_call(
        _copy_pipeline,
        out_shape=jax.ShapeDtypeStruct((_ROWS, _COLS), pos_table.dtype),
        in_specs=[pl.BlockSpec(memory_space=pl.ANY)],
        out_specs=pl.BlockSpec(memory_space=pl.ANY),
        scratch_shapes=[pltpu.VMEM((_ROWS, _COLS), jnp.float32)]
        + [pltpu.SemaphoreType.DMA] * (2 * _N_CHUNKS),
    )(pos_table)


# speedup vs baseline: 1.1229x; 1.1229x over previous
import jax
import jax.numpy as jnp
from jax.experimental import pallas as pl
from jax.experimental.pallas import tpu as pltpu

_ROWS = 8192
_COLS = 128
_N_CHUNKS = 4
_CHUNK = _ROWS // _N_CHUNKS


def _copy_pipeline(t_hbm, o_hbm, buf, *sems):
    in_sems = sems[:_N_CHUNKS]
    out_sems = sems[_N_CHUNKS:]
    ins = [
        pltpu.make_async_copy(
            t_hbm.at[pl.ds(c * _CHUNK, _CHUNK), :],
            buf.at[pl.ds(c * _CHUNK, _CHUNK), :],
            in_sems[c],
        )
        for c in range(_N_CHUNKS)
    ]
    outs = [
        pltpu.make_async_copy(
            buf.at[pl.ds(c * _CHUNK, _CHUNK), :],
            o_hbm.at[pl.ds(c * _CHUNK, _CHUNK), :],
            out_sems[c],
        )
        for c in range(_N_CHUNKS)
    ]
    for c in range(_N_CHUNKS):
        ins[c].start()
    for c in range(_N_CHUNKS):
        ins[c].wait()
        outs[c].start()
    for c in range(_N_CHUNKS):
        outs[c].wait()


def kernel(x, pos_table):
    del x
    return pl.pallas_call(
        _copy_pipeline,
        out_shape=jax.ShapeDtypeStruct((_ROWS, _COLS), pos_table.dtype),
        in_specs=[pl.BlockSpec(memory_space=pl.ANY)],
        out_specs=pl.BlockSpec(memory_space=pl.ANY),
        scratch_shapes=[pltpu.VMEM((_ROWS, _COLS), jnp.float32)]
        + [pltpu.SemaphoreType.DMA] * (2 * _N_CHUNKS),
    )(pos_table)
